# baseline (device time: 45195 ns/iter reference)
import jax
import jax.numpy as jnp
from jax import lax
from jax.experimental import pallas as pl
from jax.experimental.pallas import tpu as pltpu

B, S, H, Dh, Dr = 2, 256, 16, 64, 32
D = 1024
DC_SH = 64
NCHUNK = 4
CH = S // NCHUNK


def _dot(a, b):
    return jnp.dot(a, b, preferred_element_type=jnp.float32)


def _dot_t(a, b):
    return lax.dot_general(
        a, b, (((1,), (1,)), ((), ())), preferred_element_type=jnp.float32
    )


def kernel(x, Wdkv, Wuk, Wuv, Wq, Wqr, Wkr, Wo):
    def body(
        x_ref, wdkv_ref, wuk_ref, wuv_ref, wq_ref, wqr_ref, wkr_ref, wo_ref,
        out_ref,
        c_buf, c_rem, wuk_rem, wuv_rem, o_buf, out_buf, yrcv,
        xsend_sems, xrecv_sems, ysend_sems, yrecv_sems,
    ):
        my_x = lax.axis_index("x")
        my_y = lax.axis_index("y")
        xnbr = (1 - my_x, my_y)
        ynbr = (my_x, 1 - my_y)

        barrier_sem = pltpu.get_barrier_semaphore()
        for nbr in (xnbr, ynbr):
            pl.semaphore_signal(
                barrier_sem, inc=1, device_id=nbr,
                device_id_type=pl.DeviceIdType.MESH,
            )
        pl.semaphore_wait(barrier_sem, 2)

        xb = x_ref[my_y]
        c_buf[:] = _dot(xb, wdkv_ref[:])

        x_rdmas = []
        for i, (src, dst) in enumerate(
            [(c_buf, c_rem), (wuk_ref, wuk_rem), (wuv_ref, wuv_rem)]
        ):
            r = pltpu.make_async_remote_copy(
                src_ref=src, dst_ref=dst,
                send_sem=xsend_sems.at[i], recv_sem=xrecv_sems.at[i],
                device_id=xnbr, device_id_type=pl.DeviceIdType.MESH,
            )
            r.start()
            x_rdmas.append(r)

        scale = (Dh + Dr) ** -0.5
        q = _dot(xb, wq_ref[:]) * scale
        qr = _dot(xb, wqr_ref[:]) * scale
        kr = _dot(xb, wkr_ref[:])

        for r in x_rdmas:
            r.wait()

        k = _dot(c_buf[:], wuk_ref[:]) + _dot(c_rem[:], wuk_rem[:])
        v = _dot(c_buf[:], wuv_ref[:]) + _dot(c_rem[:], wuv_rem[:])

        for h in range(H):
            q_h = q[:, h * Dh:(h + 1) * Dh]
            k_h = k[:, h * Dh:(h + 1) * Dh]
            qr_h = qr[:, h * Dr:(h + 1) * Dr]
            s = _dot_t(q_h, k_h) + _dot_t(qr_h, kr)
            p = jnp.exp(s)
            p = p / jnp.sum(p, axis=-1, keepdims=True)
            o_buf[:, h * Dh:(h + 1) * Dh] = _dot(p, v[:, h * Dh:(h + 1) * Dh])

        y_rdmas = []
        for i in range(NCHUNK):
            sl = pl.ds(i * CH, CH)
            out_buf[sl, :] = _dot(o_buf[sl, :], wo_ref[:])
            r = pltpu.make_async_remote_copy(
                src_ref=out_buf.at[sl],
                dst_ref=yrcv.at[sl],
                send_sem=ysend_sems.at[i], recv_sem=yrecv_sems.at[i],
                device_id=ynbr, device_id_type=pl.DeviceIdType.MESH,
            )
            r.start()
            y_rdmas.append(r)

        out_ref[my_y] = out_buf[:]
        for r in y_rdmas:
            r.wait()
        out_ref[1 - my_y] = yrcv[:]

    return pl.pallas_call(
        body,
        out_shape=jax.ShapeDtypeStruct((B, S, D), jnp.float32),
        in_specs=[pl.BlockSpec(memory_space=pltpu.VMEM)] * 8,
        out_specs=pl.BlockSpec(memory_space=pltpu.VMEM),
        scratch_shapes=[
            pltpu.VMEM((S, DC_SH), jnp.float32),
            pltpu.VMEM((S, DC_SH), jnp.float32),
            pltpu.VMEM((DC_SH, D), jnp.float32),
            pltpu.VMEM((DC_SH, D), jnp.float32),
            pltpu.VMEM((S, H * Dh), jnp.float32),
            pltpu.VMEM((S, D), jnp.float32),
            pltpu.VMEM((S, D), jnp.float32),
            pltpu.SemaphoreType.DMA((3,)),
            pltpu.SemaphoreType.DMA((3,)),
            pltpu.SemaphoreType.DMA((NCHUNK,)),
            pltpu.SemaphoreType.DMA((NCHUNK,)),
        ],
        compiler_params=pltpu.CompilerParams(collective_id=0),
    )(x, Wdkv, Wuk, Wuv, Wq, Wqr, Wkr, Wo)
